# baseline (speedup 1.0000x reference)
import functools

import jax
import jax.numpy as jnp
from jax import lax
from jax.experimental import pallas as pl
from jax.experimental.pallas import tpu as pltpu
from jax.experimental.pallas import tpu_sc as plsc

N = 10000
E = 320000
NC = 2
NS = 16
NW = NC * NS
EPW = E // NW
K = 100
CH = EPW // K
RPT = N // NS


NB = 2


def _make_segsum(D, const_table=False):
  mesh = plsc.VectorSubcoreMesh(core_axis_name="c", subcore_axis_name="s",
                                num_cores=NC, num_subcores=NS)

  @functools.partial(
      pl.kernel,
      out_type=jax.ShapeDtypeStruct((NC, N, D), jnp.float32),
      mesh=mesh,
      scratch_types=[
          pltpu.VMEM((CH, K), jnp.int32),
          pltpu.VMEM((CH, K), jnp.int32),
          [pltpu.VMEM((K, D), jnp.float32) for _ in range(NB)],
          pltpu.VMEM_SHARED((N, D), jnp.float32),
          [pltpu.SemaphoreType.DMA for _ in range(NB)],
      ],
      compiler_params=pltpu.CompilerParams(use_tc_tiling_on_sc=False),
  )
  def seg(table, gidx, sidx, zeros, out, gbuf, sbuf, rows, acc, sems):
    c = lax.axis_index("c")
    s = lax.axis_index("s")
    w = c * NS + s
    pltpu.sync_copy(zeros, acc.at[pl.ds(s * RPT, RPT)])
    pltpu.sync_copy(gidx.at[w], gbuf)
    pltpu.sync_copy(sidx.at[w], sbuf)
    plsc.subcore_barrier()

    if const_table:
      pltpu.async_copy(table.at[gbuf.at[0]], rows[0], sems[0]).wait()

      def body(i, carry):
        j0 = i * NB
        for b in range(NB):
          pltpu.async_copy(rows[0], acc.at[sbuf.at[j0 + b]], sems[1], add=True)
        for b in range(NB):
          pltpu.make_async_copy(rows[0], acc.at[sbuf.at[j0]], sems[1]).wait()
        return carry

      lax.fori_loop(0, CH // NB, body, 0)
    else:
      for b in range(NB):
        pltpu.async_copy(table.at[gbuf.at[b]], rows[b], sems[b])

      def body(i, carry):
        j0 = i * NB
        for b in range(NB):
          j = j0 + b
          pltpu.make_async_copy(table.at[gbuf.at[j]], rows[b], sems[b]).wait()
          pltpu.sync_copy(rows[b], acc.at[sbuf.at[j]], add=True)
          pltpu.async_copy(table.at[gbuf.at[j + NB]], rows[b], sems[b])
        return carry

      lax.fori_loop(0, CH // NB - 1, body, 0)
      for b in range(NB):
        j = CH - NB + b
        pltpu.make_async_copy(table.at[gbuf.at[j]], rows[b], sems[b]).wait()
        pltpu.sync_copy(rows[b], acc.at[sbuf.at[j]], add=True)

    plsc.subcore_barrier()
    pltpu.sync_copy(acc.at[pl.ds(s * RPT, RPT)],
                    out.at[c].at[pl.ds(s * RPT, RPT)])

  return seg


_segsum128 = _make_segsum(128)
_segsum8_ones = _make_segsum(8, const_table=True)

NBO = 2


def _make_edge_outer():
  mesh = plsc.VectorSubcoreMesh(core_axis_name="c", subcore_axis_name="s",
                                num_cores=NC, num_subcores=NS)

  @functools.partial(
      pl.kernel,
      out_type=jax.ShapeDtypeStruct((NC, NS, 16), jnp.float32),
      mesh=mesh,
      scratch_types=[
          pltpu.VMEM((CH, K), jnp.int32),
          pltpu.VMEM((CH, K), jnp.int32),
          [pltpu.VMEM((K, 16), jnp.float32) for _ in range(NBO)],
          [pltpu.VMEM((K, 16), jnp.float32) for _ in range(NBO)],
          pltpu.VMEM((1, 16), jnp.float32),
          [pltpu.SemaphoreType.DMA for _ in range(NBO)],
          [pltpu.SemaphoreType.DMA for _ in range(NBO)],
      ],
      compiler_params=pltpu.CompilerParams(use_tc_tiling_on_sc=False),
  )
  def na(ts, td, gidx, sidx, out, gbuf, sbuf, srows, drows, accbuf, gsems,
         dsems):
    c = lax.axis_index("c")
    s = lax.axis_index("s")
    w = c * NS + s
    pltpu.sync_copy(gidx.at[w], gbuf)
    pltpu.sync_copy(sidx.at[w], sbuf)
    for b in range(NBO):
      pltpu.async_copy(ts.at[gbuf.at[b]], srows[b], gsems[b])
      pltpu.async_copy(td.at[sbuf.at[b]], drows[b], dsems[b])

    def chunk_sum(b, acc):
      def ebody(k4, a):
        k = k4 * 4
        m0 = srows[b][k] * drows[b][k]
        m1 = srows[b][k + 1] * drows[b][k + 1]
        m2 = srows[b][k + 2] * drows[b][k + 2]
        m3 = srows[b][k + 3] * drows[b][k + 3]
        return a + ((m0 + m1) + (m2 + m3))

      return lax.fori_loop(0, K // 4, ebody, acc)

    def body(i, acc):
      j0 = i * NBO
      for b in range(NBO):
        j = j0 + b
        pltpu.make_async_copy(ts.at[gbuf.at[j]], srows[b], gsems[b]).wait()
        pltpu.make_async_copy(td.at[sbuf.at[j]], drows[b], dsems[b]).wait()
        acc = chunk_sum(b, acc)
        pltpu.async_copy(ts.at[gbuf.at[j + NBO]], srows[b], gsems[b])
        pltpu.async_copy(td.at[sbuf.at[j + NBO]], drows[b], dsems[b])
      return acc

    acc = lax.fori_loop(0, CH // NBO - 1, body, jnp.zeros((16,), jnp.float32))
    for b in range(NBO):
      j = CH - NBO + b
      pltpu.make_async_copy(ts.at[gbuf.at[j]], srows[b], gsems[b]).wait()
      pltpu.make_async_copy(td.at[sbuf.at[j]], drows[b], dsems[b]).wait()
      acc = chunk_sum(b, acc)
    accbuf[0] = acc
    pltpu.sync_copy(accbuf, out.at[c].at[pl.ds(s, 1)])

  return na


_edge_outer = _make_edge_outer()


R = 1000
G = N // R


def _dinv_block(deg_ref):
  d = deg_ref[0][:, 0:1] + deg_ref[1][:, 0:1] + 1.0
  return lax.rsqrt(jnp.maximum(d, 1e-12))


def _tc1_body(f_ref, w_ref, deg_ref, out_ref):
  dinv = _dinv_block(deg_ref)
  out_ref[...] = jnp.dot(f_ref[...], w_ref[...],
                         preferred_element_type=jnp.float32) * dinv


def _tc2_body(q_ref, p_ref, deg_ref, b_ref, w_ref, out_ref):
  dinv = _dinv_block(deg_ref)
  q = q_ref[0] + q_ref[1] + p_ref[...]
  x1 = jnp.maximum(q * dinv + b_ref[...], 0.0)
  out_ref[...] = jnp.dot(x1, w_ref[...],
                         preferred_element_type=jnp.float32) * dinv


def _tc3_body(q_ref, p_ref, deg_ref, b2_ref, w3_ref, b3_ref, w4_ref, b4_ref,
              ts_ref, td_ref, gf_ref):
  i = pl.program_id(0)
  dinv = _dinv_block(deg_ref)
  nf2 = (q_ref[0] + q_ref[1] + p_ref[...]) * dinv + b2_ref[...]
  t = jnp.tanh(jnp.dot(nf2, w3_ref[...],
                       preferred_element_type=jnp.float32) + b3_ref[...])
  l8 = jnp.dot(t, w4_ref[...], preferred_element_type=jnp.float32) + b4_ref[...]
  m = jnp.max(l8, axis=1, keepdims=True)
  e = jnp.exp(l8 - m)
  a8 = e / jnp.sum(e, axis=1, keepdims=True)
  a0 = a8[:, 0:1]
  a1 = a8[:, 1:2]
  z12 = jnp.zeros((a8.shape[0], 12), jnp.float32)
  ts_ref[...] = jnp.concatenate([a0, a0, a1, a1, z12], axis=1)
  td_ref[...] = jnp.concatenate([a0, a1, a0, a1, z12], axis=1)
  contrib = lax.dot_general(a8, nf2, (((0,), (0,)), ((), ())))

  @pl.when(i == 0)
  def _():
    gf_ref[...] = contrib

  @pl.when(i > 0)
  def _():
    gf_ref[...] += contrib


def _tc4_body(na_ref, gf_ref, ge_ref, po_ref, ne_ref, pen_ref):
  t = jnp.sum(na_ref[0] + na_ref[1], axis=0, keepdims=True)
  gf = gf_ref[...]
  ge_ref[...] = (gf[0:1] + gf[1:2]) * 0.5
  po_ref[...] = jnp.clip(gf[0:1], -100.0, 100.0)
  ne_ref[...] = jnp.clip(gf[1:2], -100.0, 100.0)
  li = lax.broadcasted_iota(jnp.int32, (1, 16), 1)

  def lane(k):
    return jnp.sum(jnp.where(li == k, t, 0.0))

  n00, n01, n10, n11 = lane(0), lane(1), lane(2), lane(3)
  rn0 = jnp.maximum(jnp.abs(n00) + jnp.abs(n01), 1e-12)
  rn1 = jnp.maximum(jnp.abs(n10) + jnp.abs(n11), 1e-12)
  pen = ((n00 / rn0 - 1.0) ** 2 + (n11 / rn1 - 1.0) ** 2) * 0.5
  pen_ref[...] = pen[None, None]


def _f32(*shape):
  return jax.ShapeDtypeStruct(shape, jnp.float32)


def kernel(features, edges, W1, b1, W2, b2, W3, b3, W4, b4):
  src = edges[0].reshape(NW, CH, K)
  dst = edges[1].reshape(NW, CH, K)
  ones_tbl = jnp.ones((N, 8), jnp.float32)
  zeros8 = jnp.zeros((RPT, 8), jnp.float32)
  zeros128 = jnp.zeros((RPT, 128), jnp.float32)
  b1r = b1.reshape(1, 128)
  b2r = b2.reshape(1, 128)
  b3r = b3.reshape(1, 64)
  w4p = jnp.concatenate([W4, jnp.zeros((64, 6), jnp.float32)], axis=1)
  b4p = jnp.concatenate([b4, jnp.full((6,), -1e30, jnp.float32)]).reshape(1, 8)

  deg8 = _segsum8_ones(ones_tbl, src, dst, zeros8)

  p1 = pl.pallas_call(
      _tc1_body,
      grid=(G,),
      in_specs=[
          pl.BlockSpec((R, 128), lambda i: (i, 0)),
          pl.BlockSpec((128, 128), lambda i: (0, 0)),
          pl.BlockSpec((2, R, 8), lambda i: (0, i, 0)),
      ],
      out_specs=pl.BlockSpec((R, 128), lambda i: (i, 0)),
      out_shape=_f32(N, 128),
  )(features, W1, deg8)

  q1 = _segsum128(p1, src, dst, zeros128)

  p2 = pl.pallas_call(
      _tc2_body,
      grid=(G,),
      in_specs=[
          pl.BlockSpec((2, R, 128), lambda i: (0, i, 0)),
          pl.BlockSpec((R, 128), lambda i: (i, 0)),
          pl.BlockSpec((2, R, 8), lambda i: (0, i, 0)),
          pl.BlockSpec((1, 128), lambda i: (0, 0)),
          pl.BlockSpec((128, 128), lambda i: (0, 0)),
      ],
      out_specs=pl.BlockSpec((R, 128), lambda i: (i, 0)),
      out_shape=_f32(N, 128),
  )(q1, p1, deg8, b1r, W2)

  q2 = _segsum128(p2, src, dst, zeros128)

  ts, td, gf8 = pl.pallas_call(
      _tc3_body,
      grid=(G,),
      in_specs=[
          pl.BlockSpec((2, R, 128), lambda i: (0, i, 0)),
          pl.BlockSpec((R, 128), lambda i: (i, 0)),
          pl.BlockSpec((2, R, 8), lambda i: (0, i, 0)),
          pl.BlockSpec((1, 128), lambda i: (0, 0)),
          pl.BlockSpec((128, 64), lambda i: (0, 0)),
          pl.BlockSpec((1, 64), lambda i: (0, 0)),
          pl.BlockSpec((64, 8), lambda i: (0, 0)),
          pl.BlockSpec((1, 8), lambda i: (0, 0)),
      ],
      out_specs=[
          pl.BlockSpec((R, 16), lambda i: (i, 0)),
          pl.BlockSpec((R, 16), lambda i: (i, 0)),
          pl.BlockSpec((8, 128), lambda i: (0, 0)),
      ],
      out_shape=[_f32(N, 16), _f32(N, 16), _f32(8, 128)],
  )(q2, p2, deg8, b2r, W3, b3r, w4p, b4p)

  na = _edge_outer(ts, td, src, dst)

  ge, po, ne, pen = pl.pallas_call(
      _tc4_body,
      grid=(1,),
      in_specs=[
          pl.BlockSpec((2, 16, 16), lambda i: (0, 0, 0)),
          pl.BlockSpec((8, 128), lambda i: (0, 0)),
      ],
      out_specs=[
          pl.BlockSpec((1, 128), lambda i: (0, 0)),
          pl.BlockSpec((1, 128), lambda i: (0, 0)),
          pl.BlockSpec((1, 128), lambda i: (0, 0)),
          pl.BlockSpec((1, 1), lambda i: (0, 0)),
      ],
      out_shape=[_f32(1, 128), _f32(1, 128), _f32(1, 128), _f32(1, 1)],
  )(na, gf8)

  return ge, po, ne, pen[0, 0]


# speedup vs baseline: 1.0874x; 1.0874x over previous
import functools

import jax
import jax.numpy as jnp
from jax import lax
from jax.experimental import pallas as pl
from jax.experimental.pallas import tpu as pltpu
from jax.experimental.pallas import tpu_sc as plsc

N = 10000
E = 320000
NC = 2
NS = 16
NW = NC * NS
EPW = E // NW
K = 100
CH = EPW // K
RPT = N // NS


def _make_segsum(D, NB, const_table=False):
  mesh = plsc.VectorSubcoreMesh(core_axis_name="c", subcore_axis_name="s",
                                num_cores=NC, num_subcores=NS)

  @functools.partial(
      pl.kernel,
      out_type=jax.ShapeDtypeStruct((NC, N, D), jnp.float32),
      mesh=mesh,
      scratch_types=[
          pltpu.VMEM((CH, K), jnp.int32),
          pltpu.VMEM((CH, K), jnp.int32),
          [pltpu.VMEM((K, D), jnp.float32) for _ in range(NB)],
          pltpu.VMEM_SHARED((N, D), jnp.float32),
          [pltpu.SemaphoreType.DMA for _ in range(NB)],
      ],
      compiler_params=pltpu.CompilerParams(use_tc_tiling_on_sc=False),
  )
  def seg(table, gidx, sidx, zeros, out, gbuf, sbuf, rows, acc, sems):
    c = lax.axis_index("c")
    s = lax.axis_index("s")
    w = c * NS + s
    pltpu.sync_copy(zeros, acc.at[pl.ds(s * RPT, RPT)])
    pltpu.sync_copy(gidx.at[w], gbuf)
    pltpu.sync_copy(sidx.at[w], sbuf)
    plsc.subcore_barrier()

    if const_table:
      pltpu.async_copy(table.at[gbuf.at[0]], rows[0], sems[0]).wait()

      def body(i, carry):
        j0 = i * NB
        for b in range(NB):
          pltpu.async_copy(rows[0], acc.at[sbuf.at[j0 + b]], sems[1], add=True)
        for b in range(NB):
          pltpu.make_async_copy(rows[0], acc.at[sbuf.at[j0]], sems[1]).wait()
        return carry

      lax.fori_loop(0, CH // NB, body, 0)
    else:
      for b in range(NB):
        pltpu.async_copy(table.at[gbuf.at[b]], rows[b], sems[b])

      def body(i, carry):
        j0 = i * NB
        for b in range(NB):
          j = j0 + b
          pltpu.make_async_copy(table.at[gbuf.at[j]], rows[b], sems[b]).wait()
          pltpu.sync_copy(rows[b], acc.at[sbuf.at[j]], add=True)
          pltpu.async_copy(table.at[gbuf.at[j + NB]], rows[b], sems[b])
        return carry

      lax.fori_loop(0, CH // NB - 1, body, 0)
      for b in range(NB):
        j = CH - NB + b
        pltpu.make_async_copy(table.at[gbuf.at[j]], rows[b], sems[b]).wait()
        pltpu.sync_copy(rows[b], acc.at[sbuf.at[j]], add=True)

    plsc.subcore_barrier()
    pltpu.sync_copy(acc.at[pl.ds(s * RPT, RPT)],
                    out.at[c].at[pl.ds(s * RPT, RPT)])

  return seg


_segsum128 = _make_segsum(128, NB=2)
_segsum8 = _make_segsum(8, NB=10)
_segsum8_ones = _make_segsum(8, NB=10, const_table=True)


R = 1000
G = N // R


def _dinv_block(deg_ref):
  d = deg_ref[0][:, 0:1] + deg_ref[1][:, 0:1] + 1.0
  return lax.rsqrt(jnp.maximum(d, 1e-12))


def _tc1_body(f_ref, w_ref, deg_ref, out_ref):
  dinv = _dinv_block(deg_ref)
  out_ref[...] = jnp.dot(f_ref[...], w_ref[...],
                         preferred_element_type=jnp.float32) * dinv


def _tc2_body(q_ref, p_ref, deg_ref, b_ref, w_ref, out_ref):
  dinv = _dinv_block(deg_ref)
  q = q_ref[0] + q_ref[1] + p_ref[...]
  x1 = jnp.maximum(q * dinv + b_ref[...], 0.0)
  out_ref[...] = jnp.dot(x1, w_ref[...],
                         preferred_element_type=jnp.float32) * dinv


def _tc3_body(q_ref, p_ref, deg_ref, b2_ref, w3_ref, b3_ref, w4_ref, b4_ref,
              a8_ref, gf_ref):
  i = pl.program_id(0)
  dinv = _dinv_block(deg_ref)
  nf2 = (q_ref[0] + q_ref[1] + p_ref[...]) * dinv + b2_ref[...]
  t = jnp.tanh(jnp.dot(nf2, w3_ref[...],
                       preferred_element_type=jnp.float32) + b3_ref[...])
  l8 = jnp.dot(t, w4_ref[...], preferred_element_type=jnp.float32) + b4_ref[...]
  m = jnp.max(l8, axis=1, keepdims=True)
  e = jnp.exp(l8 - m)
  a8 = e / jnp.sum(e, axis=1, keepdims=True)
  a8_ref[...] = a8
  contrib = lax.dot_general(a8, nf2, (((0,), (0,)), ((), ())))

  @pl.when(i == 0)
  def _():
    gf_ref[...] = contrib

  @pl.when(i > 0)
  def _():
    gf_ref[...] += contrib


def _tc4_body(a8_ref, m8_ref, gf_ref, ge_ref, po_ref, ne_ref, pen_ref, acc_ref):
  i = pl.program_id(0)

  @pl.when(i == 0)
  def _():
    acc_ref[...] = jnp.zeros((8, 8), jnp.float32)

  msum = m8_ref[0] + m8_ref[1]
  acc_ref[...] += lax.dot_general(a8_ref[...], msum, (((0,), (0,)), ((), ())))

  @pl.when(i == G - 1)
  def _():
    na = acc_ref[...]
    gf = gf_ref[...]
    ge_ref[...] = (gf[0:1] + gf[1:2]) * 0.5
    po_ref[...] = jnp.clip(gf[0:1], -100.0, 100.0)
    ne_ref[...] = jnp.clip(gf[1:2], -100.0, 100.0)
    rn = jnp.maximum(jnp.sum(jnp.abs(na), axis=1, keepdims=True), 1e-12)
    rows_i = lax.broadcasted_iota(jnp.int32, (8, 8), 0)
    cols_i = lax.broadcasted_iota(jnp.int32, (8, 8), 1)
    nd = jnp.sum(jnp.where(rows_i == cols_i, na / rn, 0.0),
                 axis=1, keepdims=True)
    sq = jnp.where(lax.broadcasted_iota(jnp.int32, (8, 1), 0) < 2,
                   (nd - 1.0) ** 2, 0.0)
    pen_ref[...] = (jnp.sum(sq) * 0.5)[None, None]


def _f32(*shape):
  return jax.ShapeDtypeStruct(shape, jnp.float32)


def kernel(features, edges, W1, b1, W2, b2, W3, b3, W4, b4):
  src = edges[0].reshape(NW, CH, K)
  dst = edges[1].reshape(NW, CH, K)
  ones_tbl = jnp.ones((N, 8), jnp.float32)
  zeros8 = jnp.zeros((RPT, 8), jnp.float32)
  zeros128 = jnp.zeros((RPT, 128), jnp.float32)
  b1r = b1.reshape(1, 128)
  b2r = b2.reshape(1, 128)
  b3r = b3.reshape(1, 64)
  w4p = jnp.concatenate([W4, jnp.zeros((64, 6), jnp.float32)], axis=1)
  b4p = jnp.concatenate([b4, jnp.full((6,), -1e30, jnp.float32)]).reshape(1, 8)

  deg8 = _segsum8_ones(ones_tbl, src, dst, zeros8)

  p1 = pl.pallas_call(
      _tc1_body,
      grid=(G,),
      in_specs=[
          pl.BlockSpec((R, 128), lambda i: (i, 0)),
          pl.BlockSpec((128, 128), lambda i: (0, 0)),
          pl.BlockSpec((2, R, 8), lambda i: (0, i, 0)),
      ],
      out_specs=pl.BlockSpec((R, 128), lambda i: (i, 0)),
      out_shape=_f32(N, 128),
  )(features, W1, deg8)

  q1 = _segsum128(p1, src, dst, zeros128)

  p2 = pl.pallas_call(
      _tc2_body,
      grid=(G,),
      in_specs=[
          pl.BlockSpec((2, R, 128), lambda i: (0, i, 0)),
          pl.BlockSpec((R, 128), lambda i: (i, 0)),
          pl.BlockSpec((2, R, 8), lambda i: (0, i, 0)),
          pl.BlockSpec((1, 128), lambda i: (0, 0)),
          pl.BlockSpec((128, 128), lambda i: (0, 0)),
      ],
      out_specs=pl.BlockSpec((R, 128), lambda i: (i, 0)),
      out_shape=_f32(N, 128),
  )(q1, p1, deg8, b1r, W2)

  q2 = _segsum128(p2, src, dst, zeros128)

  a8, gf8 = pl.pallas_call(
      _tc3_body,
      grid=(G,),
      in_specs=[
          pl.BlockSpec((2, R, 128), lambda i: (0, i, 0)),
          pl.BlockSpec((R, 128), lambda i: (i, 0)),
          pl.BlockSpec((2, R, 8), lambda i: (0, i, 0)),
          pl.BlockSpec((1, 128), lambda i: (0, 0)),
          pl.BlockSpec((128, 64), lambda i: (0, 0)),
          pl.BlockSpec((1, 64), lambda i: (0, 0)),
          pl.BlockSpec((64, 8), lambda i: (0, 0)),
          pl.BlockSpec((1, 8), lambda i: (0, 0)),
      ],
      out_specs=[
          pl.BlockSpec((R, 8), lambda i: (i, 0)),
          pl.BlockSpec((8, 128), lambda i: (0, 0)),
      ],
      out_shape=[_f32(N, 8), _f32(8, 128)],
  )(q2, p2, deg8, b2r, W3, b3r, w4p, b4p)

  m8 = _segsum8(a8, dst, src, zeros8)

  ge, po, ne, pen = pl.pallas_call(
      _tc4_body,
      grid=(G,),
      in_specs=[
          pl.BlockSpec((R, 8), lambda i: (i, 0)),
          pl.BlockSpec((2, R, 8), lambda i: (0, i, 0)),
          pl.BlockSpec((8, 128), lambda i: (0, 0)),
      ],
      out_specs=[
          pl.BlockSpec((1, 128), lambda i: (0, 0)),
          pl.BlockSpec((1, 128), lambda i: (0, 0)),
          pl.BlockSpec((1, 128), lambda i: (0, 0)),
          pl.BlockSpec((1, 1), lambda i: (0, 0)),
      ],
      out_shape=[_f32(1, 128), _f32(1, 128), _f32(1, 128), _f32(1, 1)],
      scratch_shapes=[pltpu.VMEM((8, 8), jnp.float32)],
  )(a8, m8, gf8)

  return ge, po, ne, pen[0, 0]


# speedup vs baseline: 1.0999x; 1.0115x over previous
import functools

import jax
import jax.numpy as jnp
from jax import lax
from jax.experimental import pallas as pl
from jax.experimental.pallas import tpu as pltpu
from jax.experimental.pallas import tpu_sc as plsc

N = 10000
E = 320000
NC = 2
NS = 16
NW = NC * NS
EPW = E // NW
K = 100
CH = EPW // K
RPT = N // NS


def _make_segsum(D, NB, const_table=False):
  mesh = plsc.VectorSubcoreMesh(core_axis_name="c", subcore_axis_name="s",
                                num_cores=NC, num_subcores=NS)

  @functools.partial(
      pl.kernel,
      out_type=jax.ShapeDtypeStruct((NC, N, D), jnp.float32),
      mesh=mesh,
      scratch_types=[
          pltpu.VMEM((CH, K), jnp.int32),
          pltpu.VMEM((CH, K), jnp.int32),
          [pltpu.VMEM((K, D), jnp.float32) for _ in range(NB)],
          pltpu.VMEM_SHARED((N, D), jnp.float32),
          [pltpu.SemaphoreType.DMA for _ in range(NB)],
      ],
      compiler_params=pltpu.CompilerParams(use_tc_tiling_on_sc=False),
  )
  def seg(table, gidx, sidx, zeros, out, gbuf, sbuf, rows, acc, sems):
    c = lax.axis_index("c")
    s = lax.axis_index("s")
    w = c * NS + s
    pltpu.sync_copy(gidx.at[w], gbuf)
    pltpu.sync_copy(sidx.at[w], sbuf)
    if const_table:
      pltpu.async_copy(table.at[gbuf.at[0]], rows[0], sems[0])
    else:
      for b in range(NB):
        pltpu.async_copy(table.at[gbuf.at[b]], rows[b], sems[b])
    pltpu.sync_copy(zeros, acc.at[pl.ds(s * RPT, RPT)])
    plsc.subcore_barrier()

    if const_table:
      pltpu.make_async_copy(table.at[gbuf.at[0]], rows[0], sems[0]).wait()

      def body(i, carry):
        j0 = i * NB
        for b in range(NB):
          pltpu.async_copy(rows[0], acc.at[sbuf.at[j0 + b]], sems[1], add=True)
        for b in range(NB):
          pltpu.make_async_copy(rows[0], acc.at[sbuf.at[j0]], sems[1]).wait()
        return carry

      lax.fori_loop(0, CH // NB, body, 0)
    else:
      def body(i, carry):
        j0 = i * NB
        for b in range(NB):
          j = j0 + b
          pltpu.make_async_copy(table.at[gbuf.at[j]], rows[b], sems[b]).wait()
          pltpu.sync_copy(rows[b], acc.at[sbuf.at[j]], add=True)
          pltpu.async_copy(table.at[gbuf.at[j + NB]], rows[b], sems[b])
        return carry

      lax.fori_loop(0, CH // NB - 1, body, 0)
      for b in range(NB):
        j = CH - NB + b
        pltpu.make_async_copy(table.at[gbuf.at[j]], rows[b], sems[b]).wait()
        pltpu.sync_copy(rows[b], acc.at[sbuf.at[j]], add=True)

    plsc.subcore_barrier()
    pltpu.sync_copy(acc.at[pl.ds(s * RPT, RPT)],
                    out.at[c].at[pl.ds(s * RPT, RPT)])

  return seg


_segsum128 = _make_segsum(128, NB=2)
_segsum8 = _make_segsum(8, NB=10)
_segsum8_ones = _make_segsum(8, NB=10, const_table=True)


R = 1000
G = N // R


def _dinv_block(deg_ref):
  d = deg_ref[0][:, 0:1] + deg_ref[1][:, 0:1] + 1.0
  return lax.rsqrt(jnp.maximum(d, 1e-12))


def _tc0_body(f_ref, w_ref, out_ref):
  out_ref[...] = jnp.dot(f_ref[...], w_ref[...],
                         preferred_element_type=jnp.float32)


def _tc1_body(h_ref, deg_ref, out_ref):
  out_ref[...] = h_ref[...] * _dinv_block(deg_ref)


def _tc2_body(q_ref, p_ref, deg_ref, b_ref, w_ref, out_ref):
  dinv = _dinv_block(deg_ref)
  q = q_ref[0] + q_ref[1] + p_ref[...]
  x1 = jnp.maximum(q * dinv + b_ref[...], 0.0)
  out_ref[...] = jnp.dot(x1, w_ref[...],
                         preferred_element_type=jnp.float32) * dinv


def _tc3_body(q_ref, p_ref, deg_ref, b2_ref, w3_ref, b3_ref, w4_ref, b4_ref,
              a8_ref, gf_ref):
  i = pl.program_id(0)
  dinv = _dinv_block(deg_ref)
  nf2 = (q_ref[0] + q_ref[1] + p_ref[...]) * dinv + b2_ref[...]
  t = jnp.tanh(jnp.dot(nf2, w3_ref[...],
                       preferred_element_type=jnp.float32) + b3_ref[...])
  l8 = jnp.dot(t, w4_ref[...], preferred_element_type=jnp.float32) + b4_ref[...]
  m = jnp.max(l8, axis=1, keepdims=True)
  e = jnp.exp(l8 - m)
  a8 = e / jnp.sum(e, axis=1, keepdims=True)
  a8_ref[...] = a8
  contrib = lax.dot_general(a8, nf2, (((0,), (0,)), ((), ())))

  @pl.when(i == 0)
  def _():
    gf_ref[...] = contrib

  @pl.when(i > 0)
  def _():
    gf_ref[...] += contrib


def _tc4_body(a8_ref, m8_ref, gf_ref, ge_ref, po_ref, ne_ref, pen_ref, acc_ref):
  i = pl.program_id(0)

  @pl.when(i == 0)
  def _():
    acc_ref[...] = jnp.zeros((8, 8), jnp.float32)

  msum = m8_ref[0] + m8_ref[1]
  acc_ref[...] += lax.dot_general(a8_ref[...], msum, (((0,), (0,)), ((), ())))

  @pl.when(i == G - 1)
  def _():
    na = acc_ref[...]
    gf = gf_ref[...]
    ge_ref[...] = (gf[0:1] + gf[1:2]) * 0.5
    po_ref[...] = jnp.clip(gf[0:1], -100.0, 100.0)
    ne_ref[...] = jnp.clip(gf[1:2], -100.0, 100.0)
    rn = jnp.maximum(jnp.sum(jnp.abs(na), axis=1, keepdims=True), 1e-12)
    rows_i = lax.broadcasted_iota(jnp.int32, (8, 8), 0)
    cols_i = lax.broadcasted_iota(jnp.int32, (8, 8), 1)
    nd = jnp.sum(jnp.where(rows_i == cols_i, na / rn, 0.0),
                 axis=1, keepdims=True)
    sq = jnp.where(lax.broadcasted_iota(jnp.int32, (8, 1), 0) < 2,
                   (nd - 1.0) ** 2, 0.0)
    pen_ref[...] = (jnp.sum(sq) * 0.5)[None, None]


def _f32(*shape):
  return jax.ShapeDtypeStruct(shape, jnp.float32)


def kernel(features, edges, W1, b1, W2, b2, W3, b3, W4, b4):
  src = edges[0].reshape(NW, CH, K)
  dst = edges[1].reshape(NW, CH, K)
  ones_tbl = jnp.ones((N, 8), jnp.float32)
  zeros8 = jnp.zeros((RPT, 8), jnp.float32)
  zeros128 = jnp.zeros((RPT, 128), jnp.float32)
  b1r = b1.reshape(1, 128)
  b2r = b2.reshape(1, 128)
  b3r = b3.reshape(1, 64)
  w4p = jnp.concatenate([W4, jnp.zeros((64, 6), jnp.float32)], axis=1)
  b4p = jnp.concatenate([b4, jnp.full((6,), -1e30, jnp.float32)]).reshape(1, 8)

  deg8 = _segsum8_ones(ones_tbl, src, dst, zeros8)

  h1 = pl.pallas_call(
      _tc0_body,
      grid=(G,),
      in_specs=[
          pl.BlockSpec((R, 128), lambda i: (i, 0)),
          pl.BlockSpec((128, 128), lambda i: (0, 0)),
      ],
      out_specs=pl.BlockSpec((R, 128), lambda i: (i, 0)),
      out_shape=_f32(N, 128),
  )(features, W1)

  p1 = pl.pallas_call(
      _tc1_body,
      grid=(G,),
      in_specs=[
          pl.BlockSpec((R, 128), lambda i: (i, 0)),
          pl.BlockSpec((2, R, 8), lambda i: (0, i, 0)),
      ],
      out_specs=pl.BlockSpec((R, 128), lambda i: (i, 0)),
      out_shape=_f32(N, 128),
  )(h1, deg8)

  q1 = _segsum128(p1, src, dst, zeros128)

  p2 = pl.pallas_call(
      _tc2_body,
      grid=(G,),
      in_specs=[
          pl.BlockSpec((2, R, 128), lambda i: (0, i, 0)),
          pl.BlockSpec((R, 128), lambda i: (i, 0)),
          pl.BlockSpec((2, R, 8), lambda i: (0, i, 0)),
          pl.BlockSpec((1, 128), lambda i: (0, 0)),
          pl.BlockSpec((128, 128), lambda i: (0, 0)),
      ],
      out_specs=pl.BlockSpec((R, 128), lambda i: (i, 0)),
      out_shape=_f32(N, 128),
  )(q1, p1, deg8, b1r, W2)

  q2 = _segsum128(p2, src, dst, zeros128)

  a8, gf8 = pl.pallas_call(
      _tc3_body,
      grid=(G,),
      in_specs=[
          pl.BlockSpec((2, R, 128), lambda i: (0, i, 0)),
          pl.BlockSpec((R, 128), lambda i: (i, 0)),
          pl.BlockSpec((2, R, 8), lambda i: (0, i, 0)),
          pl.BlockSpec((1, 128), lambda i: (0, 0)),
          pl.BlockSpec((128, 64), lambda i: (0, 0)),
          pl.BlockSpec((1, 64), lambda i: (0, 0)),
          pl.BlockSpec((64, 8), lambda i: (0, 0)),
          pl.BlockSpec((1, 8), lambda i: (0, 0)),
      ],
      out_specs=[
          pl.BlockSpec((R, 8), lambda i: (i, 0)),
          pl.BlockSpec((8, 128), lambda i: (0, 0)),
      ],
      out_shape=[_f32(N, 8), _f32(8, 128)],
  )(q2, p2, deg8, b2r, W3, b3r, w4p, b4p)

  m8 = _segsum8(a8, dst, src, zeros8)

  ge, po, ne, pen = pl.pallas_call(
      _tc4_body,
      grid=(G,),
      in_specs=[
          pl.BlockSpec((R, 8), lambda i: (i, 0)),
          pl.BlockSpec((2, R, 8), lambda i: (0, i, 0)),
          pl.BlockSpec((8, 128), lambda i: (0, 0)),
      ],
      out_specs=[
          pl.BlockSpec((1, 128), lambda i: (0, 0)),
          pl.BlockSpec((1, 128), lambda i: (0, 0)),
          pl.BlockSpec((1, 128), lambda i: (0, 0)),
          pl.BlockSpec((1, 1), lambda i: (0, 0)),
      ],
      out_shape=[_f32(1, 128), _f32(1, 128), _f32(1, 128), _f32(1, 1)],
      scratch_shapes=[pltpu.VMEM((8, 8), jnp.float32)],
  )(a8, m8, gf8)

  return ge, po, ne, pen[0, 0]


# speedup vs baseline: 1.1025x; 1.0024x over previous
import functools

import jax
import jax.numpy as jnp
from jax import lax
from jax.experimental import pallas as pl
from jax.experimental.pallas import tpu as pltpu
from jax.experimental.pallas import tpu_sc as plsc

N = 10000
E = 320000
NC = 2
NS = 16
NW = NC * NS
EPW = E // NW
K = 100
CH = EPW // K
RPT = N // NS


def _make_segsum(D, NB, const_table=False):
  mesh = plsc.VectorSubcoreMesh(core_axis_name="c", subcore_axis_name="s",
                                num_cores=NC, num_subcores=NS)

  @functools.partial(
      pl.kernel,
      out_type=jax.ShapeDtypeStruct((NC, N, D), jnp.float32),
      mesh=mesh,
      scratch_types=[
          pltpu.VMEM((CH, K), jnp.int32),
          pltpu.VMEM((CH, K), jnp.int32),
          [pltpu.VMEM((K, D), jnp.float32) for _ in range(NB)],
          pltpu.VMEM_SHARED((N, D), jnp.float32),
          [pltpu.SemaphoreType.DMA for _ in range(NB)],
      ],
      compiler_params=pltpu.CompilerParams(use_tc_tiling_on_sc=False),
  )
  def seg(table, gidx, sidx, zeros, out, gbuf, sbuf, rows, acc, sems):
    c = lax.axis_index("c")
    s = lax.axis_index("s")
    w = c * NS + s
    pltpu.sync_copy(gidx.at[w], gbuf)
    pltpu.sync_copy(sidx.at[w], sbuf)
    if const_table:
      pltpu.async_copy(table.at[gbuf.at[0]], rows[0], sems[0])
    else:
      for b in range(NB):
        pltpu.async_copy(table.at[gbuf.at[b]], rows[b], sems[b])
    pltpu.sync_copy(zeros, acc.at[pl.ds(s * RPT, RPT)])
    plsc.subcore_barrier()

    if const_table:
      pltpu.make_async_copy(table.at[gbuf.at[0]], rows[0], sems[0]).wait()

      def body(i, carry):
        j0 = i * NB
        for b in range(NB):
          pltpu.async_copy(rows[0], acc.at[sbuf.at[j0 + b]], sems[1], add=True)
        for b in range(NB):
          pltpu.make_async_copy(rows[0], acc.at[sbuf.at[j0]], sems[1]).wait()
        return carry

      lax.fori_loop(0, CH // NB, body, 0)
    else:
      def body(i, carry):
        j0 = i * NB
        for b in range(NB):
          j = j0 + b
          pltpu.make_async_copy(table.at[gbuf.at[j]], rows[b], sems[b]).wait()
          pltpu.sync_copy(rows[b], acc.at[sbuf.at[j]], add=True)
          pltpu.async_copy(table.at[gbuf.at[j + NB]], rows[b], sems[b])
        return carry

      lax.fori_loop(0, CH // NB - 1, body, 0)
      for b in range(NB):
        j = CH - NB + b
        pltpu.make_async_copy(table.at[gbuf.at[j]], rows[b], sems[b]).wait()
        pltpu.sync_copy(rows[b], acc.at[sbuf.at[j]], add=True)

    plsc.subcore_barrier()
    pltpu.sync_copy(acc.at[pl.ds(s * RPT, RPT)],
                    out.at[c].at[pl.ds(s * RPT, RPT)])

  return seg


_segsum128 = _make_segsum(128, NB=2)
_segsum8 = _make_segsum(8, NB=10)
_segsum8_ones = _make_segsum(8, NB=10, const_table=True)


R = 1000
G = N // R


def _dinv_block(deg_ref):
  d = deg_ref[0][:, 0:1] + deg_ref[1][:, 0:1] + 1.0
  return lax.rsqrt(jnp.maximum(d, 1e-12))


def _tc1_body(f_ref, w_ref, deg_ref, out_ref):
  dinv = _dinv_block(deg_ref)
  out_ref[...] = jnp.dot(f_ref[...], w_ref[...],
                         preferred_element_type=jnp.float32) * dinv


def _tc2_body(q_ref, p_ref, deg_ref, b_ref, w_ref, out_ref):
  dinv = _dinv_block(deg_ref)
  q = q_ref[0] + q_ref[1] + p_ref[...]
  x1 = jnp.maximum(q * dinv + b_ref[...], 0.0)
  out_ref[...] = jnp.dot(x1, w_ref[...],
                         preferred_element_type=jnp.float32) * dinv


def _tc3_body(q_ref, p_ref, deg_ref, b2_ref, w3_ref, b3_ref, w4_ref, b4_ref,
              a8_ref, gf_ref):
  i = pl.program_id(0)
  dinv = _dinv_block(deg_ref)
  nf2 = (q_ref[0] + q_ref[1] + p_ref[...]) * dinv + b2_ref[...]
  t = jnp.tanh(jnp.dot(nf2, w3_ref[...],
                       preferred_element_type=jnp.float32) + b3_ref[...])
  l8 = jnp.dot(t, w4_ref[...], preferred_element_type=jnp.float32) + b4_ref[...]
  m = jnp.max(l8, axis=1, keepdims=True)
  e = jnp.exp(l8 - m)
  a8 = e / jnp.sum(e, axis=1, keepdims=True)
  a8_ref[...] = a8
  contrib = lax.dot_general(a8, nf2, (((0,), (0,)), ((), ())))

  @pl.when(i == 0)
  def _():
    gf_ref[...] = contrib

  @pl.when(i > 0)
  def _():
    gf_ref[...] += contrib


def _tc4_body(a8_ref, m8_ref, gf_ref, ge_ref, po_ref, ne_ref, pen_ref, acc_ref):
  i = pl.program_id(0)

  @pl.when(i == 0)
  def _():
    acc_ref[...] = jnp.zeros((8, 8), jnp.float32)

  msum = m8_ref[0] + m8_ref[1]
  acc_ref[...] += lax.dot_general(a8_ref[...], msum, (((0,), (0,)), ((), ())))

  @pl.when(i == G - 1)
  def _():
    na = acc_ref[...]
    gf = gf_ref[...]
    ge_ref[...] = (gf[0:1] + gf[1:2]) * 0.5
    po_ref[...] = jnp.clip(gf[0:1], -100.0, 100.0)
    ne_ref[...] = jnp.clip(gf[1:2], -100.0, 100.0)
    rn = jnp.maximum(jnp.sum(jnp.abs(na), axis=1, keepdims=True), 1e-12)
    rows_i = lax.broadcasted_iota(jnp.int32, (8, 8), 0)
    cols_i = lax.broadcasted_iota(jnp.int32, (8, 8), 1)
    nd = jnp.sum(jnp.where(rows_i == cols_i, na / rn, 0.0),
                 axis=1, keepdims=True)
    sq = jnp.where(lax.broadcasted_iota(jnp.int32, (8, 1), 0) < 2,
                   (nd - 1.0) ** 2, 0.0)
    pen_ref[...] = (jnp.sum(sq) * 0.5)[None, None]


def _f32(*shape):
  return jax.ShapeDtypeStruct(shape, jnp.float32)


def kernel(features, edges, W1, b1, W2, b2, W3, b3, W4, b4):
  src = edges[0].reshape(NW, CH, K)
  dst = edges[1].reshape(NW, CH, K)
  ones_tbl = jnp.ones((N, 8), jnp.float32)
  zeros8 = jnp.zeros((RPT, 8), jnp.float32)
  zeros128 = jnp.zeros((RPT, 128), jnp.float32)
  b1r = b1.reshape(1, 128)
  b2r = b2.reshape(1, 128)
  b3r = b3.reshape(1, 64)
  w4p = jnp.concatenate([W4, jnp.zeros((64, 6), jnp.float32)], axis=1)
  b4p = jnp.concatenate([b4, jnp.full((6,), -1e30, jnp.float32)]).reshape(1, 8)

  deg8 = _segsum8_ones(ones_tbl, src, dst, zeros8)

  p1 = pl.pallas_call(
      _tc1_body,
      grid=(G,),
      in_specs=[
          pl.BlockSpec((R, 128), lambda i: (i, 0)),
          pl.BlockSpec((128, 128), lambda i: (0, 0)),
          pl.BlockSpec((2, R, 8), lambda i: (0, i, 0)),
      ],
      out_specs=pl.BlockSpec((R, 128), lambda i: (i, 0)),
      out_shape=_f32(N, 128),
  )(features, W1, deg8)

  q1 = _segsum128(p1, src, dst, zeros128)

  p2 = pl.pallas_call(
      _tc2_body,
      grid=(G,),
      in_specs=[
          pl.BlockSpec((2, R, 128), lambda i: (0, i, 0)),
          pl.BlockSpec((R, 128), lambda i: (i, 0)),
          pl.BlockSpec((2, R, 8), lambda i: (0, i, 0)),
          pl.BlockSpec((1, 128), lambda i: (0, 0)),
          pl.BlockSpec((128, 128), lambda i: (0, 0)),
      ],
      out_specs=pl.BlockSpec((R, 128), lambda i: (i, 0)),
      out_shape=_f32(N, 128),
  )(q1, p1, deg8, b1r, W2)

  q2 = _segsum128(p2, src, dst, zeros128)

  a8, gf8 = pl.pallas_call(
      _tc3_body,
      grid=(G,),
      in_specs=[
          pl.BlockSpec((2, R, 128), lambda i: (0, i, 0)),
          pl.BlockSpec((R, 128), lambda i: (i, 0)),
          pl.BlockSpec((2, R, 8), lambda i: (0, i, 0)),
          pl.BlockSpec((1, 128), lambda i: (0, 0)),
          pl.BlockSpec((128, 64), lambda i: (0, 0)),
          pl.BlockSpec((1, 64), lambda i: (0, 0)),
          pl.BlockSpec((64, 8), lambda i: (0, 0)),
          pl.BlockSpec((1, 8), lambda i: (0, 0)),
      ],
      out_specs=[
          pl.BlockSpec((R, 8), lambda i: (i, 0)),
          pl.BlockSpec((8, 128), lambda i: (0, 0)),
      ],
      out_shape=[_f32(N, 8), _f32(8, 128)],
  )(q2, p2, deg8, b2r, W3, b3r, w4p, b4p)

  m8 = _segsum8(a8, dst, src, zeros8)

  ge, po, ne, pen = pl.pallas_call(
      _tc4_body,
      grid=(G,),
      in_specs=[
          pl.BlockSpec((R, 8), lambda i: (i, 0)),
          pl.BlockSpec((2, R, 8), lambda i: (0, i, 0)),
          pl.BlockSpec((8, 128), lambda i: (0, 0)),
      ],
      out_specs=[
          pl.BlockSpec((1, 128), lambda i: (0, 0)),
          pl.BlockSpec((1, 128), lambda i: (0, 0)),
          pl.BlockSpec((1, 128), lambda i: (0, 0)),
          pl.BlockSpec((1, 1), lambda i: (0, 0)),
      ],
      out_shape=[_f32(1, 128), _f32(1, 128), _f32(1, 128), _f32(1, 1)],
      scratch_shapes=[pltpu.VMEM((8, 8), jnp.float32)],
  )(a8, m8, gf8)

  return ge, po, ne, pen[0, 0]
